# baseline (speedup 1.0000x reference)
import functools

import jax
import jax.numpy as jnp
from jax.experimental import pallas as pl


def _lrelu(a):
    return jnp.maximum(a, 0.01 * a)


def _dotT(lhsT, rhs):
    return jax.lax.dot_general(lhsT, rhs, (((0,), (0,)), ((), ())))


def _mpn_kernel(x_ref, ei_ref, ea_ref, u_ref,
                W_ne_ref, b_ne_ref, W_ee_ref, b_ee_ref, W_ge_ref, b_ge_ref,
                W_eu_ref, b_eu_ref, W_nu_ref, b_nu_ref, W_eu2_ref, b_eu2_ref,
                Ws_mat_ref, b_s_ref, out_ref, *, N, E, D, BLK):
    xg = x_ref[0]
    src = ei_ref[0, 0, :]
    dst = ei_ref[0, 1, :]
    eg = ea_ref[0]
    ug = u_ref[0]

    f32 = jnp.float32

    g_enc = _lrelu(jnp.dot(ug, W_ge_ref[...]) + b_ge_ref[...])
    e_enc = _lrelu(jnp.dot(eg, W_ee_ref[...]) + b_ee_ref[...])

    W_ne = W_ne_ref[...]
    A = jnp.dot(xg, W_ne[:D]) + b_ne_ref[...]

    iota_n = jax.lax.broadcasted_iota(jnp.int32, (N, E), 0)
    GsT = (src[None, :] == iota_n).astype(f32)
    GdT = (dst[None, :] == iota_n).astype(f32)
    GsdT = jnp.concatenate([GsT, GdT], axis=1)

    A_sd = _dotT(GsdT, A)
    A_src, A_dst = A_sd[:E], A_sd[E:]

    W_eu = W_eu_ref[...]
    C_e1 = jnp.dot(e_enc, W_eu[2 * D:3 * D]) + jnp.dot(g_enc, W_eu[3 * D:]) + b_eu_ref[...]
    W_nu = W_nu_ref[...]
    C_n2 = jnp.dot(g_enc, W_nu[2 * D:]) + b_nu_ref[...]
    W_eu2 = W_eu2_ref[...]
    C_e2 = jnp.dot(g_enc, W_eu2[3 * D:]) + b_eu2_ref[...]

    R = jnp.dot(xg, W_ne[D:])

    Acat = jnp.concatenate([A_src, A_dst], axis=1)
    Rcat = jnp.concatenate([R, R], axis=1)
    X1 = _lrelu(Acat[None, :, :] + Rcat[:, None, :]).reshape(BLK * E, 2 * D)
    e1 = _lrelu(jnp.dot(X1, W_eu[:2 * D]).reshape(BLK, E, D)
                + C_e1[None, :, :])

    agg = jnp.concatenate(
        [jnp.dot(GdT, e1[j])[None] for j in range(BLK)], axis=0)

    n_nodes = _lrelu(A[None, :, :] + R[:, None, :])
    ncat = jnp.concatenate([n_nodes, agg], axis=-1)
    n2 = _lrelu(jnp.dot(ncat.reshape(BLK * N, 2 * D), W_nu[:2 * D])
                + C_n2).reshape(BLK, N, D)

    n2r = n2.reshape(BLK * N, D)
    P = jnp.dot(n2r, W_eu2[:D]).reshape(BLK, N, D)
    Q = jnp.dot(n2r, W_eu2[D:2 * D]).reshape(BLK, N, D)
    PQ = jnp.concatenate([P, Q], axis=1)
    e1W = jnp.dot(e1.reshape(BLK * E, D), W_eu2[2 * D:3 * D])

    iota_e = jax.lax.broadcasted_iota(jnp.int32, (E, 2 * N), 1)
    Gsd2 = ((iota_e == src[:, None])
            | (iota_e == dst[:, None] + N)).astype(f32)

    sd = jnp.concatenate(
        [jnp.dot(Gsd2, PQ[j])[None] for j in range(BLK)], axis=0)
    e2 = _lrelu(sd.reshape(BLK * E, D) + e1W + C_e2)

    OUT = jnp.dot(e2, Ws_mat_ref[...]).reshape(BLK, E, BLK)
    bi = jax.lax.broadcasted_iota(jnp.int32, (BLK, 1, BLK), 0)
    li = jax.lax.broadcasted_iota(jnp.int32, (BLK, 1, BLK), 2)
    msk = (bi == li).astype(f32)
    out_ref[0] = jnp.sum(OUT * msk, axis=0) + b_s_ref[0, 0]


@jax.jit
def kernel(x, edge_index, edge_attr, u, W_ne, b_ne, W_ee, b_ee, W_ge, b_ge,
           W_eu, b_eu, W_nu, b_nu, W_eu2, b_eu2, W_s, b_s):
    B, N, D = x.shape
    E = edge_attr.shape[1]
    BLK = 64

    row = lambda v: v.reshape(1, -1)
    Ws_mat = jnp.broadcast_to(W_s, (D, BLK))
    b_s2 = b_s.reshape(1, 1)

    full = lambda a: pl.BlockSpec(a.shape, lambda b, t: (0,) * a.ndim)

    out = pl.pallas_call(
        functools.partial(_mpn_kernel, N=N, E=E, D=D, BLK=BLK),
        grid=(B, N // BLK),
        in_specs=[
            pl.BlockSpec((1, N, D), lambda b, t: (b, 0, 0)),
            pl.BlockSpec((1, 2, E), lambda b, t: (b, 0, 0)),
            pl.BlockSpec((1, E, D), lambda b, t: (b, 0, 0)),
            pl.BlockSpec((1, 1, D), lambda b, t: (b, 0, 0)),
            full(W_ne), full(row(b_ne)), full(W_ee), full(row(b_ee)),
            full(W_ge), full(row(b_ge)), full(W_eu), full(row(b_eu)),
            full(W_nu), full(row(b_nu)), full(W_eu2), full(row(b_eu2)),
            full(Ws_mat), full(b_s2),
        ],
        out_specs=pl.BlockSpec((1, E, BLK), lambda b, t: (b, 0, t)),
        out_shape=jax.ShapeDtypeStruct((B, E, N), jnp.float32),
    )(x, edge_index, edge_attr, u.reshape(B, 1, D),
      W_ne, row(b_ne), W_ee, row(b_ee), W_ge, row(b_ge),
      W_eu, row(b_eu), W_nu, row(b_nu), W_eu2, row(b_eu2),
      Ws_mat, b_s2)

    return out.reshape(-1)


# speedup vs baseline: 1.1006x; 1.1006x over previous
import functools

import jax
import jax.numpy as jnp
from jax.experimental import pallas as pl


def _lrelu(a):
    return jnp.maximum(a, 0.01 * a)


def _dotT(lhsT, rhs):
    return jax.lax.dot_general(lhsT, rhs, (((0,), (0,)), ((), ())))


def _mpn_kernel(x_ref, ei_ref, ea_ref, u_ref,
                W_ne_ref, b_ne_ref, W_ee_ref, b_ee_ref, W_ge_ref, b_ge_ref,
                W_eu_ref, b_eu_ref, W_nu_ref, b_nu_ref, W_eu2_ref, b_eu2_ref,
                Ws_row_ref, b_s_ref, out_ref, *, N, E, D, BLK):
    xg = x_ref[0]
    src = ei_ref[0, 0, :]
    dst = ei_ref[0, 1, :]
    eg = ea_ref[0]
    ug = u_ref[0]

    f32 = jnp.float32

    g_enc = _lrelu(jnp.dot(ug, W_ge_ref[...]) + b_ge_ref[...])
    e_enc = _lrelu(jnp.dot(eg, W_ee_ref[...]) + b_ee_ref[...])

    W_ne = W_ne_ref[...]
    A = jnp.dot(xg, W_ne[:D]) + b_ne_ref[...]

    iota_n = jax.lax.broadcasted_iota(jnp.int32, (N, E), 0)
    GsT = (src[None, :] == iota_n).astype(f32)
    GdT = (dst[None, :] == iota_n).astype(f32)
    GsdT = jnp.concatenate([GsT, GdT], axis=1)

    A_sd = _dotT(GsdT, A)
    A_src, A_dst = A_sd[:E], A_sd[E:]

    W_eu = W_eu_ref[...]
    C_e1 = jnp.dot(e_enc, W_eu[2 * D:3 * D]) + jnp.dot(g_enc, W_eu[3 * D:]) + b_eu_ref[...]
    W_nu = W_nu_ref[...]
    C_n2 = jnp.dot(g_enc, W_nu[2 * D:]) + b_nu_ref[...]
    W_eu2 = W_eu2_ref[...]
    C_e2 = jnp.dot(g_enc, W_eu2[3 * D:]) + b_eu2_ref[...]

    R = jnp.dot(xg, W_ne[D:])

    Acat = jnp.concatenate([A_src, A_dst], axis=1)
    Rcat = jnp.concatenate([R, R], axis=1)
    X1 = _lrelu(Acat[None, :, :] + Rcat[:, None, :]).reshape(BLK * E, 2 * D)
    e1 = _lrelu(jnp.dot(X1, W_eu[:2 * D]).reshape(BLK, E, D)
                + C_e1[None, :, :])

    agg = jnp.concatenate(
        [jnp.dot(GdT, e1[j])[None] for j in range(BLK)], axis=0)

    n_nodes = _lrelu(A[None, :, :] + R[:, None, :])
    ncat = jnp.concatenate([n_nodes, agg], axis=-1)
    n2 = _lrelu(jnp.dot(ncat.reshape(BLK * N, 2 * D), W_nu[:2 * D])
                + C_n2).reshape(BLK, N, D)

    n2r = n2.reshape(BLK * N, D)
    P = jnp.dot(n2r, W_eu2[:D]).reshape(BLK, N, D)
    Q = jnp.dot(n2r, W_eu2[D:2 * D]).reshape(BLK, N, D)
    PQ = jnp.concatenate([P, Q], axis=1)
    e1W = jnp.dot(e1.reshape(BLK * E, D), W_eu2[2 * D:3 * D])

    iota_e = jax.lax.broadcasted_iota(jnp.int32, (E, 2 * N), 1)
    Gsd2 = ((iota_e == src[:, None])
            | (iota_e == dst[:, None] + N)).astype(f32)

    sd = jnp.concatenate(
        [jnp.dot(Gsd2, PQ[j])[None] for j in range(BLK)], axis=0)
    e2 = _lrelu(sd.reshape(BLK * E, D) + e1W + C_e2)

    sc = jnp.sum(e2.reshape(BLK, E, D) * Ws_row_ref[...][None, :, :], axis=-1)
    out_ref[0] = sc.T + b_s_ref[0, 0]


@jax.jit
def kernel(x, edge_index, edge_attr, u, W_ne, b_ne, W_ee, b_ee, W_ge, b_ge,
           W_eu, b_eu, W_nu, b_nu, W_eu2, b_eu2, W_s, b_s):
    B, N, D = x.shape
    E = edge_attr.shape[1]
    BLK = 64

    row = lambda v: v.reshape(1, -1)
    Ws_row = W_s.T
    b_s2 = b_s.reshape(1, 1)

    full = lambda a: pl.BlockSpec(a.shape, lambda b, t: (0,) * a.ndim)

    out = pl.pallas_call(
        functools.partial(_mpn_kernel, N=N, E=E, D=D, BLK=BLK),
        grid=(B, N // BLK),
        in_specs=[
            pl.BlockSpec((1, N, D), lambda b, t: (b, 0, 0)),
            pl.BlockSpec((1, 2, E), lambda b, t: (b, 0, 0)),
            pl.BlockSpec((1, E, D), lambda b, t: (b, 0, 0)),
            pl.BlockSpec((1, 1, D), lambda b, t: (b, 0, 0)),
            full(W_ne), full(row(b_ne)), full(W_ee), full(row(b_ee)),
            full(W_ge), full(row(b_ge)), full(W_eu), full(row(b_eu)),
            full(W_nu), full(row(b_nu)), full(W_eu2), full(row(b_eu2)),
            full(Ws_row), full(b_s2),
        ],
        out_specs=pl.BlockSpec((1, E, BLK), lambda b, t: (b, 0, t)),
        out_shape=jax.ShapeDtypeStruct((B, E, N), jnp.float32),
    )(x, edge_index, edge_attr, u.reshape(B, 1, D),
      W_ne, row(b_ne), W_ee, row(b_ee), W_ge, row(b_ge),
      W_eu, row(b_eu), W_nu, row(b_nu), W_eu2, row(b_eu2),
      Ws_row, b_s2)

    return out.reshape(-1)
